# baseline (speedup 1.0000x reference)
import functools

import jax
import jax.numpy as jnp
from jax import lax
from jax.experimental import pallas as pl
from jax.experimental.pallas import tpu as pltpu
from jax.experimental.pallas import tpu_sc as plsc

NUM_EXPERTS = 8
TOP_K = 2
D_IN = 1024
D_FF = 2048
D_OUT = 1024
T = 2048

BM = 128
NT = 39
R_MM = NT * BM
R = 5120


def _gating_body(x_ref, wg_ref, bg_ref, w_ref, i_ref):
    logits = jnp.dot(x_ref[...], wg_ref[...], preferred_element_type=jnp.float32)
    logits = logits + bg_ref[...]
    m = jnp.max(logits, axis=1, keepdims=True)
    p = jnp.exp(logits - m)
    g = p / jnp.sum(p, axis=1, keepdims=True)
    iota = lax.broadcasted_iota(jnp.int32, g.shape, 1)
    w1 = jnp.max(g, axis=1, keepdims=True)
    i1 = jnp.min(jnp.where(g == w1, iota, NUM_EXPERTS), axis=1, keepdims=True)
    g2 = jnp.where(iota == i1, -1.0, g)
    w2 = jnp.max(g2, axis=1, keepdims=True)
    i2 = jnp.min(jnp.where(g2 == w2, iota, NUM_EXPERTS), axis=1, keepdims=True)
    w_ref[...] = jnp.concatenate([w1, w2], axis=1)
    i_ref[...] = jnp.concatenate([i1, i2], axis=1)


def _gating(x, Wg, bg):
    return pl.pallas_call(
        _gating_body,
        out_shape=[
            jax.ShapeDtypeStruct((T, TOP_K), jnp.float32),
            jax.ShapeDtypeStruct((T, TOP_K), jnp.int32),
        ],
    )(x, Wg, bg.reshape(1, NUM_EXPERTS))


def _gather_rows(xb, row_tok):
    info = plsc.get_sparse_core_info()
    nw = info.num_cores * info.num_subcores
    rows_pw = R // nw
    chunk = rows_pw // 2
    wrow = D_IN // 2
    mesh = plsc.VectorSubcoreMesh(core_axis_name="c", subcore_axis_name="s")

    @functools.partial(
        pl.kernel, mesh=mesh,
        out_type=jax.ShapeDtypeStruct((R, wrow), jnp.int32),
        scratch_types=[
            pltpu.VMEM((chunk,), jnp.int32),
            pltpu.VMEM((chunk,), jnp.int32),
            pltpu.VMEM((chunk, wrow), jnp.int32),
            pltpu.VMEM((chunk, wrow), jnp.int32),
            pltpu.SemaphoreType.DMA,
        ],
    )
    def k(x_hbm, tok_hbm, out_hbm, i0_v, i1_v, r0_v, r1_v, sem):
        wid = lax.axis_index("s") * info.num_cores + lax.axis_index("c")
        base = wid * rows_pw
        pltpu.sync_copy(tok_hbm.at[pl.ds(base, chunk)], i0_v)
        pltpu.sync_copy(tok_hbm.at[pl.ds(base + chunk, chunk)], i1_v)
        g0 = pltpu.async_copy(x_hbm.at[i0_v], r0_v, sem)
        g1 = pltpu.async_copy(x_hbm.at[i1_v], r1_v, sem)
        g0.wait()
        pltpu.sync_copy(r0_v, out_hbm.at[pl.ds(base, chunk)])
        g1.wait()
        pltpu.sync_copy(r1_v, out_hbm.at[pl.ds(base + chunk, chunk)])

    return k(xb, row_tok)


def _ffn_body(te_ref, xs_ref, w1_ref, b1_ref, w2_ref, b2_ref, rw_ref, out_ref):
    h = jnp.dot(xs_ref[...], w1_ref[0], preferred_element_type=jnp.float32)
    h = jnp.maximum(h + b1_ref[0], 0.0).astype(jnp.bfloat16)
    y = jnp.dot(h, w2_ref[0], preferred_element_type=jnp.float32)
    y = y + b2_ref[0]
    out_ref[...] = y * rw_ref[...]


def _expert_ffn(xs, tile_e, row_w, W1, b1, W2, b2):
    grid_spec = pltpu.PrefetchScalarGridSpec(
        num_scalar_prefetch=1,
        grid=(NT,),
        in_specs=[
            pl.BlockSpec((BM, D_IN), lambda i, te: (i, 0)),
            pl.BlockSpec((1, D_IN, D_FF), lambda i, te: (te[i], 0, 0)),
            pl.BlockSpec((1, 1, D_FF), lambda i, te: (te[i], 0, 0)),
            pl.BlockSpec((1, D_FF, D_OUT), lambda i, te: (te[i], 0, 0)),
            pl.BlockSpec((1, 1, D_OUT), lambda i, te: (te[i], 0, 0)),
            pl.BlockSpec((BM, 1), lambda i, te: (i, 0)),
        ],
        out_specs=pl.BlockSpec((BM, D_OUT), lambda i, te: (i, 0)),
    )
    return pl.pallas_call(
        _ffn_body,
        grid_spec=grid_spec,
        out_shape=jax.ShapeDtypeStruct((R_MM, D_OUT), jnp.float32),
        compiler_params=pltpu.CompilerParams(
            dimension_semantics=("arbitrary",)),
    )(tile_e, xs, W1, b1, W2, b2, row_w)


def _combine(ys, pos0, pos1):
    info = plsc.get_sparse_core_info()
    nw = info.num_cores * info.num_subcores
    toks_pw = T // nw
    chunk = toks_pw // 2
    mesh = plsc.VectorSubcoreMesh(core_axis_name="c", subcore_axis_name="s")

    @functools.partial(
        pl.kernel, mesh=mesh,
        out_type=jax.ShapeDtypeStruct((T, D_OUT), jnp.float32),
        scratch_types=[
            pltpu.VMEM((chunk,), jnp.int32),
            pltpu.VMEM((chunk,), jnp.int32),
            pltpu.VMEM((chunk, D_OUT), jnp.float32),
            pltpu.VMEM((chunk, D_OUT), jnp.float32),
            pltpu.SemaphoreType.DMA,
        ],
    )
    def k(ys_hbm, p0_hbm, p1_hbm, out_hbm, i0_v, i1_v, r0_v, r1_v, sem):
        wid = lax.axis_index("s") * info.num_cores + lax.axis_index("c")
        for ci in range(2):
            base = wid * toks_pw + ci * chunk
            pltpu.sync_copy(p0_hbm.at[pl.ds(base, chunk)], i0_v)
            pltpu.sync_copy(p1_hbm.at[pl.ds(base, chunk)], i1_v)
            pltpu.async_copy(ys_hbm.at[i0_v], r0_v, sem).wait()
            pltpu.async_copy(ys_hbm.at[i1_v], r1_v, sem).wait()

            def row_add(r, _):
                for c in range(D_OUT // 16):
                    sl = pl.ds(c * 16, 16)
                    r0_v[r, sl] = r0_v[r, sl] + r1_v[r, sl]
                return 0

            lax.fori_loop(0, chunk, row_add, 0)
            pltpu.sync_copy(r0_v, out_hbm.at[pl.ds(base, chunk)])

    return k(ys, pos0, pos1)


def kernel(x, Wg, bg, W1, b1, W2, b2):
    w, eidx = _gating(x, Wg, bg)

    flat_e = eidx.reshape(-1)
    flat_w = w.reshape(-1)
    flat_tok = jnp.arange(T * TOP_K, dtype=jnp.int32) // TOP_K
    order = jnp.argsort(flat_e)
    sorted_e = flat_e[order]
    counts = jnp.sum(
        (flat_e[:, None] == jnp.arange(NUM_EXPERTS, dtype=flat_e.dtype)[None, :]
         ).astype(jnp.int32), axis=0)
    padded = ((counts + BM - 1) // BM) * BM
    seg_end = jnp.cumsum(padded)
    seg_start = seg_end - padded
    sort_start = jnp.cumsum(counts) - counts
    rank = jnp.arange(T * TOP_K, dtype=jnp.int32) - sort_start[sorted_e]
    dst = (seg_start[sorted_e] + rank).astype(jnp.int32)
    row_tok = jnp.zeros((R,), jnp.int32).at[dst].set(flat_tok[order])
    row_w = jnp.zeros((R,), jnp.float32).at[dst].set(flat_w[order])
    pos = jnp.zeros((T * TOP_K,), jnp.int32).at[order].set(dst).reshape(T, TOP_K)
    tile_start = jnp.arange(NT, dtype=jnp.int32) * BM
    tile_e = jnp.sum((tile_start[:, None] >= seg_end[None, :]).astype(jnp.int32),
                     axis=1)
    tile_e = jnp.minimum(tile_e, NUM_EXPERTS - 1).astype(jnp.int32)

    xb = lax.bitcast_convert_type(
        x.astype(jnp.bfloat16).reshape(T, D_IN // 2, 2), jnp.int32)
    xs = lax.bitcast_convert_type(
        _gather_rows(xb, row_tok), jnp.bfloat16
    ).reshape(R, D_IN)
    ys = _expert_ffn(xs[:R_MM], tile_e, row_w[:R_MM, None],
                     W1.astype(jnp.bfloat16),
                     b1.reshape(NUM_EXPERTS, 1, D_FF),
                     W2.astype(jnp.bfloat16),
                     b2.reshape(NUM_EXPERTS, 1, D_OUT))
    out = _combine(ys, pos[:, 0], pos[:, 1])
    return out


# speedup vs baseline: 1.2835x; 1.2835x over previous
import functools

import jax
import jax.numpy as jnp
from jax import lax
from jax.experimental import pallas as pl
from jax.experimental.pallas import tpu as pltpu
from jax.experimental.pallas import tpu_sc as plsc

NUM_EXPERTS = 8
TOP_K = 2
D_IN = 1024
D_FF = 2048
D_OUT = 1024
T = 2048

BM = 128
NT = 39
R_MM = NT * BM
R = 5120


def _gating_body(x_ref, wg_ref, bg_ref, w_ref, i_ref):
    logits = jnp.dot(x_ref[...], wg_ref[...], preferred_element_type=jnp.float32)
    logits = logits + bg_ref[...]
    m = jnp.max(logits, axis=1, keepdims=True)
    p = jnp.exp(logits - m)
    g = p / jnp.sum(p, axis=1, keepdims=True)
    iota = lax.broadcasted_iota(jnp.int32, g.shape, 1)
    w1 = jnp.max(g, axis=1, keepdims=True)
    i1 = jnp.min(jnp.where(g == w1, iota, NUM_EXPERTS), axis=1, keepdims=True)
    g2 = jnp.where(iota == i1, -1.0, g)
    w2 = jnp.max(g2, axis=1, keepdims=True)
    i2 = jnp.min(jnp.where(g2 == w2, iota, NUM_EXPERTS), axis=1, keepdims=True)
    w_ref[...] = jnp.concatenate([w1, w2], axis=1)
    i_ref[...] = jnp.concatenate([i1, i2], axis=1)


def _gating(x, Wg, bg):
    return pl.pallas_call(
        _gating_body,
        out_shape=[
            jax.ShapeDtypeStruct((T, TOP_K), jnp.float32),
            jax.ShapeDtypeStruct((T, TOP_K), jnp.int32),
        ],
    )(x, Wg, bg.reshape(1, NUM_EXPERTS))


def _gather_rows(x, row_tok):
    info = plsc.get_sparse_core_info()
    nw = info.num_cores * info.num_subcores
    rows_pw = R // nw
    chunk = rows_pw // 4
    mesh = plsc.VectorSubcoreMesh(core_axis_name="c", subcore_axis_name="s")

    @functools.partial(
        pl.kernel, mesh=mesh,
        out_type=jax.ShapeDtypeStruct((R, D_IN), jnp.float32),
        scratch_types=[
            pltpu.VMEM((4, chunk), jnp.int32),
            pltpu.VMEM((chunk, D_IN), jnp.float32),
            pltpu.VMEM((chunk, D_IN), jnp.float32),
            pltpu.SemaphoreType.DMA,
            pltpu.SemaphoreType.DMA,
        ],
    )
    def k(x_hbm, tok_hbm, out_hbm, idx_v, ra_v, rb_v, sem_a, sem_b):
        wid = lax.axis_index("s") * info.num_cores + lax.axis_index("c")
        base = wid * rows_pw
        for ci in range(4):
            pltpu.sync_copy(tok_hbm.at[pl.ds(base + ci * chunk, chunk)],
                            idx_v.at[ci])
        bufs = (ra_v, rb_v)
        sems = (sem_a, sem_b)
        copies = [None, None, None, None]
        copies[0] = pltpu.async_copy(x_hbm.at[idx_v.at[0]], ra_v, sem_a)
        copies[1] = pltpu.async_copy(x_hbm.at[idx_v.at[1]], rb_v, sem_b)
        for ci in range(4):
            copies[ci].wait()
            pltpu.sync_copy(bufs[ci % 2],
                            out_hbm.at[pl.ds(base + ci * chunk, chunk)])
            if ci + 2 < 4:
                copies[ci + 2] = pltpu.async_copy(
                    x_hbm.at[idx_v.at[ci + 2]], bufs[ci % 2], sems[ci % 2])

    return k(x, row_tok)


def _ffn_body(te_ref, xs_ref, w1_ref, b1_ref, w2_ref, b2_ref, rw_ref, out_ref):
    h = jnp.dot(xs_ref[...].astype(jnp.bfloat16), w1_ref[0],
                preferred_element_type=jnp.float32)
    h = jnp.maximum(h + b1_ref[0], 0.0).astype(jnp.bfloat16)
    y = jnp.dot(h, w2_ref[0], preferred_element_type=jnp.float32)
    y = y + b2_ref[0]
    out_ref[...] = y * rw_ref[...]


def _expert_ffn(xs, tile_e, row_w, W1, b1, W2, b2):
    grid_spec = pltpu.PrefetchScalarGridSpec(
        num_scalar_prefetch=1,
        grid=(NT,),
        in_specs=[
            pl.BlockSpec((BM, D_IN), lambda i, te: (i, 0)),
            pl.BlockSpec((1, D_IN, D_FF), lambda i, te: (te[i], 0, 0)),
            pl.BlockSpec((1, 1, D_FF), lambda i, te: (te[i], 0, 0)),
            pl.BlockSpec((1, D_FF, D_OUT), lambda i, te: (te[i], 0, 0)),
            pl.BlockSpec((1, 1, D_OUT), lambda i, te: (te[i], 0, 0)),
            pl.BlockSpec((BM, 1), lambda i, te: (i, 0)),
        ],
        out_specs=pl.BlockSpec((BM, D_OUT), lambda i, te: (i, 0)),
    )
    return pl.pallas_call(
        _ffn_body,
        grid_spec=grid_spec,
        out_shape=jax.ShapeDtypeStruct((R_MM, D_OUT), jnp.float32),
        compiler_params=pltpu.CompilerParams(
            dimension_semantics=("arbitrary",)),
    )(tile_e, xs, W1, b1, W2, b2, row_w)


def _combine(ys, pos0, pos1):
    info = plsc.get_sparse_core_info()
    nw = info.num_cores * info.num_subcores
    toks_pw = T // nw
    chunk = toks_pw // 2
    mesh = plsc.VectorSubcoreMesh(core_axis_name="c", subcore_axis_name="s")

    @functools.partial(
        pl.kernel, mesh=mesh,
        out_type=jax.ShapeDtypeStruct((T, D_OUT), jnp.float32),
        scratch_types=[
            pltpu.VMEM((chunk,), jnp.int32),
            pltpu.VMEM((chunk,), jnp.int32),
            pltpu.VMEM((chunk, D_OUT), jnp.float32),
            pltpu.VMEM((chunk, D_OUT), jnp.float32),
            pltpu.SemaphoreType.DMA,
        ],
    )
    def k(ys_hbm, p0_hbm, p1_hbm, out_hbm, i0_v, i1_v, r0_v, r1_v, sem):
        wid = lax.axis_index("s") * info.num_cores + lax.axis_index("c")
        for ci in range(2):
            base = wid * toks_pw + ci * chunk
            pltpu.sync_copy(p0_hbm.at[pl.ds(base, chunk)], i0_v)
            pltpu.sync_copy(p1_hbm.at[pl.ds(base, chunk)], i1_v)
            pltpu.async_copy(ys_hbm.at[i0_v], r0_v, sem).wait()
            pltpu.async_copy(ys_hbm.at[i1_v], r1_v, sem).wait()

            def row_add(r, _):
                for c in range(D_OUT // 16):
                    sl = pl.ds(c * 16, 16)
                    r0_v[r, sl] = r0_v[r, sl] + r1_v[r, sl]
                return 0

            lax.fori_loop(0, chunk, row_add, 0)
            pltpu.sync_copy(r0_v, out_hbm.at[pl.ds(base, chunk)])

    return k(ys, pos0, pos1)


def kernel(x, Wg, bg, W1, b1, W2, b2):
    w, eidx = _gating(x, Wg, bg)

    flat_e = eidx.reshape(-1)
    flat_w = w.reshape(-1)
    flat_tok = jnp.arange(T * TOP_K, dtype=jnp.int32) // TOP_K
    order = jnp.argsort(flat_e)
    sorted_e = flat_e[order]
    counts = jnp.sum(
        (flat_e[:, None] == jnp.arange(NUM_EXPERTS, dtype=flat_e.dtype)[None, :]
         ).astype(jnp.int32), axis=0)
    padded = ((counts + BM - 1) // BM) * BM
    seg_end = jnp.cumsum(padded)
    seg_start = seg_end - padded
    sort_start = jnp.cumsum(counts) - counts
    rank = jnp.arange(T * TOP_K, dtype=jnp.int32) - sort_start[sorted_e]
    dst = (seg_start[sorted_e] + rank).astype(jnp.int32)
    row_tok = jnp.zeros((R,), jnp.int32).at[dst].set(flat_tok[order])
    row_w = jnp.zeros((R,), jnp.float32).at[dst].set(flat_w[order])
    pos = jnp.zeros((T * TOP_K,), jnp.int32).at[order].set(dst).reshape(T, TOP_K)
    tile_start = jnp.arange(NT, dtype=jnp.int32) * BM
    tile_e = jnp.sum((tile_start[:, None] >= seg_end[None, :]).astype(jnp.int32),
                     axis=1)
    tile_e = jnp.minimum(tile_e, NUM_EXPERTS - 1).astype(jnp.int32)

    xs = _gather_rows(x, row_tok)
    ys = _expert_ffn(xs[:R_MM], tile_e, row_w[:R_MM, None],
                     W1.astype(jnp.bfloat16),
                     b1.reshape(NUM_EXPERTS, 1, D_FF),
                     W2.astype(jnp.bfloat16),
                     b2.reshape(NUM_EXPERTS, 1, D_OUT))
    out = _combine(ys, pos[:, 0], pos[:, 1])
    return out


# speedup vs baseline: 2.7257x; 2.1237x over previous
import functools

import jax
import jax.numpy as jnp
from jax import lax
from jax.experimental import pallas as pl
from jax.experimental.pallas import tpu as pltpu
from jax.experimental.pallas import tpu_sc as plsc

NUM_EXPERTS = 8
TOP_K = 2
D_IN = 1024
D_FF = 2048
D_OUT = 1024
T = 2048

BM = 128
NT = 39
R_MM = NT * BM
NTE = 64


def _route_body(x_ref, wg_ref, bg_ref, w_ref, dst_ref, te_ref):
    logits = jnp.dot(x_ref[...], wg_ref[...], preferred_element_type=jnp.float32)
    logits = logits + bg_ref[...]
    m = jnp.max(logits, axis=1, keepdims=True)
    p = jnp.exp(logits - m)
    g = p / jnp.sum(p, axis=1, keepdims=True)
    iota = lax.broadcasted_iota(jnp.int32, g.shape, 1)
    w1 = jnp.max(g, axis=1, keepdims=True)
    i1 = jnp.min(jnp.where(g == w1, iota, NUM_EXPERTS), axis=1, keepdims=True)
    g2 = jnp.where(iota == i1, -1.0, g)
    w2 = jnp.max(g2, axis=1, keepdims=True)
    i2 = jnp.min(jnp.where(g2 == w2, iota, NUM_EXPERTS), axis=1, keepdims=True)
    w_ref[...] = jnp.concatenate([w1, w2], axis=1)

    oh0 = (iota == i1).astype(jnp.float32)
    oh1 = (iota == i2).astype(jnp.float32)
    oh = oh0 + oh1
    r_i = lax.broadcasted_iota(jnp.int32, (T, T), 0)
    c_i = lax.broadcasted_iota(jnp.int32, (T, T), 1)
    lt = (c_i < r_i).astype(jnp.bfloat16)
    pref = jnp.dot(lt, oh.astype(jnp.bfloat16),
                   preferred_element_type=jnp.float32)
    totals = pref[T - 1:T, :] + oh[T - 1:T, :]
    padded = jnp.floor((totals + (BM - 1)) * (1.0 / BM)) * BM
    r8 = lax.broadcasted_iota(jnp.int32, (NUM_EXPERTS, NUM_EXPERTS), 0)
    c8 = lax.broadcasted_iota(jnp.int32, (NUM_EXPERTS, NUM_EXPERTS), 1)
    lt8 = (r8 < c8).astype(jnp.float32)
    seg_start = jnp.dot(padded, lt8, preferred_element_type=jnp.float32)
    seg_end = seg_start + padded
    rank0 = jnp.sum(pref * oh0, axis=1, keepdims=True)
    rank1 = jnp.sum(pref * oh1, axis=1, keepdims=True)
    s0 = jnp.sum(seg_start * oh0, axis=1, keepdims=True)
    s1 = jnp.sum(seg_start * oh1, axis=1, keepdims=True)
    dst_ref[...] = jnp.concatenate([s0 + rank0, s1 + rank1],
                                   axis=1).astype(jnp.int32)

    ti = (lax.broadcasted_iota(jnp.int32, (NTE, NUM_EXPERTS), 0)
          * BM).astype(jnp.float32)
    over = (ti >= seg_end).astype(jnp.int32)
    te_ref[...] = jnp.minimum(jnp.sum(over, axis=1, keepdims=True),
                              NUM_EXPERTS - 1).astype(jnp.int32)


def _route(x, Wg, bg):
    return pl.pallas_call(
        _route_body,
        out_shape=[
            jax.ShapeDtypeStruct((T, TOP_K), jnp.float32),
            jax.ShapeDtypeStruct((T, TOP_K), jnp.int32),
            jax.ShapeDtypeStruct((NTE, 1), jnp.int32),
        ],
    )(x, Wg, bg.reshape(1, NUM_EXPERTS))


def _dispatch(x, d0, d1, w0, w1):
    info = plsc.get_sparse_core_info()
    nw = info.num_cores * info.num_subcores
    tpw = T // nw
    mesh = plsc.VectorSubcoreMesh(core_axis_name="c", subcore_axis_name="s")

    @functools.partial(
        pl.kernel, mesh=mesh,
        out_type=[
            jax.ShapeDtypeStruct((R_MM, D_IN), jnp.float32),
            jax.ShapeDtypeStruct((R_MM, 128), jnp.float32),
        ],
        scratch_types=[
            pltpu.VMEM((tpw, D_IN), jnp.float32),
            pltpu.VMEM((tpw,), jnp.int32),
            pltpu.VMEM((tpw,), jnp.int32),
            pltpu.VMEM((tpw, 128), jnp.float32),
            pltpu.VMEM((tpw, 128), jnp.float32),
            pltpu.SemaphoreType.DMA,
            pltpu.SemaphoreType.DMA,
        ],
    )
    def k(x_hbm, d0_hbm, d1_hbm, w0_hbm, w1_hbm, xs_hbm, rw_hbm,
          rows_v, i0_v, i1_v, wv0, wv1, sem_r, sem_w):
        wid = lax.axis_index("s") * info.num_cores + lax.axis_index("c")
        base = wid * tpw
        ld = pltpu.async_copy(x_hbm.at[pl.ds(base, tpw)], rows_v, sem_r)
        pltpu.sync_copy(d0_hbm.at[pl.ds(base, tpw)], i0_v)
        pltpu.sync_copy(d1_hbm.at[pl.ds(base, tpw)], i1_v)
        pltpu.sync_copy(w0_hbm.at[pl.ds(base, tpw)], wv0)
        pltpu.sync_copy(w1_hbm.at[pl.ds(base, tpw)], wv1)
        ld.wait()
        s0 = pltpu.async_copy(rows_v, xs_hbm.at[i0_v], sem_w)
        s1 = pltpu.async_copy(rows_v, xs_hbm.at[i1_v], sem_w)
        sw0 = pltpu.async_copy(wv0, rw_hbm.at[i0_v], sem_w)
        sw1 = pltpu.async_copy(wv1, rw_hbm.at[i1_v], sem_w)
        s0.wait()
        s1.wait()
        sw0.wait()
        sw1.wait()

    return k(x, d0, d1, w0, w1)


def _ffn_body(te_ref, xs_ref, w1_ref, b1_ref, w2_ref, b2_ref, rw_ref, out_ref):
    h = jnp.dot(xs_ref[...], w1_ref[0], preferred_element_type=jnp.float32)
    h = jnp.maximum(h + b1_ref[0], 0.0)
    y = jnp.dot(h, w2_ref[0], preferred_element_type=jnp.float32)
    y = y + b2_ref[0]
    out_ref[...] = y * rw_ref[...][:, :1]


def _expert_ffn(xs, tile_e, row_w, W1, b1, W2, b2):
    grid_spec = pltpu.PrefetchScalarGridSpec(
        num_scalar_prefetch=1,
        grid=(NT,),
        in_specs=[
            pl.BlockSpec((BM, D_IN), lambda i, te: (i, 0)),
            pl.BlockSpec((1, D_IN, D_FF), lambda i, te: (te[i], 0, 0)),
            pl.BlockSpec((1, 1, D_FF), lambda i, te: (te[i], 0, 0)),
            pl.BlockSpec((1, D_FF, D_OUT), lambda i, te: (te[i], 0, 0)),
            pl.BlockSpec((1, 1, D_OUT), lambda i, te: (te[i], 0, 0)),
            pl.BlockSpec((BM, 128), lambda i, te: (i, 0)),
        ],
        out_specs=pl.BlockSpec((BM, D_OUT), lambda i, te: (i, 0)),
    )
    return pl.pallas_call(
        _ffn_body,
        grid_spec=grid_spec,
        out_shape=jax.ShapeDtypeStruct((R_MM, D_OUT), jnp.float32),
        compiler_params=pltpu.CompilerParams(
            dimension_semantics=("arbitrary",)),
    )(tile_e, xs, W1, b1, W2, b2, row_w)


def _combine(ys, pos0, pos1):
    info = plsc.get_sparse_core_info()
    nw = info.num_cores * info.num_subcores
    toks_pw = T // nw
    chunk = toks_pw // 2
    mesh = plsc.VectorSubcoreMesh(core_axis_name="c", subcore_axis_name="s")

    @functools.partial(
        pl.kernel, mesh=mesh,
        out_type=jax.ShapeDtypeStruct((T, D_OUT), jnp.float32),
        scratch_types=[
            pltpu.VMEM((chunk,), jnp.int32),
            pltpu.VMEM((chunk,), jnp.int32),
            pltpu.VMEM((chunk, D_OUT), jnp.float32),
            pltpu.VMEM((chunk, D_OUT), jnp.float32),
            pltpu.SemaphoreType.DMA,
        ],
    )
    def k(ys_hbm, p0_hbm, p1_hbm, out_hbm, i0_v, i1_v, r0_v, r1_v, sem):
        wid = lax.axis_index("s") * info.num_cores + lax.axis_index("c")
        for ci in range(2):
            base = wid * toks_pw + ci * chunk
            pltpu.sync_copy(p0_hbm.at[pl.ds(base, chunk)], i0_v)
            pltpu.sync_copy(p1_hbm.at[pl.ds(base, chunk)], i1_v)
            g0 = pltpu.async_copy(ys_hbm.at[i0_v], r0_v, sem)
            g1 = pltpu.async_copy(ys_hbm.at[i1_v], r1_v, sem)
            g0.wait()
            g1.wait()

            def row_add(r, _):
                for c in range(D_OUT // 16):
                    sl = pl.ds(c * 16, 16)
                    r0_v[r, sl] = r0_v[r, sl] + r1_v[r, sl]
                return 0

            lax.fori_loop(0, chunk, row_add, 0)
            pltpu.sync_copy(r0_v, out_hbm.at[pl.ds(base, chunk)])

    return k(ys, pos0, pos1)


def kernel(x, Wg, bg, W1, b1, W2, b2):
    w, dst, te = _route(x, Wg, bg)
    d0 = dst[:, 0]
    d1 = dst[:, 1]
    tile_e = te.reshape(NTE)[:NT]
    w0b = jnp.broadcast_to(w[:, :1], (T, 128))
    w1b = jnp.broadcast_to(w[:, 1:], (T, 128))
    xs, rw = _dispatch(x, d0, d1, w0b, w1b)
    ys = _expert_ffn(xs, tile_e, rw,
                     W1, b1.reshape(NUM_EXPERTS, 1, D_FF),
                     W2, b2.reshape(NUM_EXPERTS, 1, D_OUT))
    out = _combine(ys, d0, d1)
    return out


# speedup vs baseline: 2.8355x; 1.0403x over previous
import functools

import jax
import jax.numpy as jnp
from jax import lax
from jax.experimental import pallas as pl
from jax.experimental.pallas import tpu as pltpu
from jax.experimental.pallas import tpu_sc as plsc

NUM_EXPERTS = 8
TOP_K = 2
D_IN = 1024
D_FF = 2048
D_OUT = 1024
T = 2048

BM = 128
NT = 39
R_MM = NT * BM
NTE = 64


def _route_body(x_ref, wg_ref, bg_ref, w_ref, w2_ref, dst_ref, te_ref):
    logits = jnp.dot(x_ref[...], wg_ref[...], preferred_element_type=jnp.float32)
    logits = logits + bg_ref[...]
    m = jnp.max(logits, axis=1, keepdims=True)
    p = jnp.exp(logits - m)
    g = p / jnp.sum(p, axis=1, keepdims=True)
    iota = lax.broadcasted_iota(jnp.int32, g.shape, 1)
    w1 = jnp.max(g, axis=1, keepdims=True)
    i1 = jnp.min(jnp.where(g == w1, iota, NUM_EXPERTS), axis=1, keepdims=True)
    g2 = jnp.where(iota == i1, -1.0, g)
    w2 = jnp.max(g2, axis=1, keepdims=True)
    i2 = jnp.min(jnp.where(g2 == w2, iota, NUM_EXPERTS), axis=1, keepdims=True)
    w_ref[...] = jnp.broadcast_to(w1, (T, 128))
    w2_ref[...] = jnp.broadcast_to(w2, (T, 128))

    oh0 = (iota == i1).astype(jnp.float32)
    oh1 = (iota == i2).astype(jnp.float32)
    oh = oh0 + oh1
    r_i = lax.broadcasted_iota(jnp.int32, (T, T), 0)
    c_i = lax.broadcasted_iota(jnp.int32, (T, T), 1)
    lt = (c_i < r_i).astype(jnp.bfloat16)
    pref = jnp.dot(lt, oh.astype(jnp.bfloat16),
                   preferred_element_type=jnp.float32)
    totals = pref[T - 1:T, :] + oh[T - 1:T, :]
    padded = jnp.floor((totals + (BM - 1)) * (1.0 / BM)) * BM
    r8 = lax.broadcasted_iota(jnp.int32, (NUM_EXPERTS, NUM_EXPERTS), 0)
    c8 = lax.broadcasted_iota(jnp.int32, (NUM_EXPERTS, NUM_EXPERTS), 1)
    lt8 = (r8 < c8).astype(jnp.float32)
    seg_start = jnp.dot(padded, lt8, preferred_element_type=jnp.float32)
    seg_end = seg_start + padded
    rank0 = jnp.sum(pref * oh0, axis=1, keepdims=True)
    rank1 = jnp.sum(pref * oh1, axis=1, keepdims=True)
    s0 = jnp.sum(seg_start * oh0, axis=1, keepdims=True)
    s1 = jnp.sum(seg_start * oh1, axis=1, keepdims=True)
    dst_ref[...] = jnp.concatenate([s0 + rank0, s1 + rank1],
                                   axis=1).astype(jnp.int32)

    ti = (lax.broadcasted_iota(jnp.int32, (NTE, NUM_EXPERTS), 0)
          * BM).astype(jnp.float32)
    over = (ti >= seg_end).astype(jnp.int32)
    te_ref[...] = jnp.minimum(jnp.sum(over, axis=1, keepdims=True),
                              NUM_EXPERTS - 1).astype(jnp.int32)


def _route(x, Wg, bg):
    return pl.pallas_call(
        _route_body,
        out_shape=[
            jax.ShapeDtypeStruct((T, 128), jnp.float32),
            jax.ShapeDtypeStruct((T, 128), jnp.float32),
            jax.ShapeDtypeStruct((T, TOP_K), jnp.int32),
            jax.ShapeDtypeStruct((NTE, 1), jnp.int32),
        ],
    )(x, Wg, bg.reshape(1, NUM_EXPERTS))


def _dispatch(x, d0, d1, w0, w1):
    info = plsc.get_sparse_core_info()
    nw = info.num_cores * info.num_subcores
    tpw = T // nw
    mesh = plsc.VectorSubcoreMesh(core_axis_name="c", subcore_axis_name="s")

    @functools.partial(
        pl.kernel, mesh=mesh,
        out_type=[
            jax.ShapeDtypeStruct((R_MM, D_IN), jnp.float32),
            jax.ShapeDtypeStruct((R_MM, 128), jnp.float32),
        ],
        scratch_types=[
            pltpu.VMEM((tpw, D_IN), jnp.float32),
            pltpu.VMEM((tpw,), jnp.int32),
            pltpu.VMEM((tpw,), jnp.int32),
            pltpu.VMEM((tpw, 128), jnp.float32),
            pltpu.VMEM((tpw, 128), jnp.float32),
            pltpu.SemaphoreType.DMA,
            pltpu.SemaphoreType.DMA,
        ],
    )
    def k(x_hbm, d0_hbm, d1_hbm, w0_hbm, w1_hbm, xs_hbm, rw_hbm,
          rows_v, i0_v, i1_v, wv0, wv1, sem_r, sem_w):
        wid = lax.axis_index("s") * info.num_cores + lax.axis_index("c")
        base = wid * tpw
        ld = pltpu.async_copy(x_hbm.at[pl.ds(base, tpw)], rows_v, sem_r)
        pltpu.sync_copy(d0_hbm.at[pl.ds(base, tpw)], i0_v)
        pltpu.sync_copy(d1_hbm.at[pl.ds(base, tpw)], i1_v)
        pltpu.sync_copy(w0_hbm.at[pl.ds(base, tpw)], wv0)
        pltpu.sync_copy(w1_hbm.at[pl.ds(base, tpw)], wv1)
        ld.wait()
        s0 = pltpu.async_copy(rows_v, xs_hbm.at[i0_v], sem_w)
        s1 = pltpu.async_copy(rows_v, xs_hbm.at[i1_v], sem_w)
        sw0 = pltpu.async_copy(wv0, rw_hbm.at[i0_v], sem_w)
        sw1 = pltpu.async_copy(wv1, rw_hbm.at[i1_v], sem_w)
        s0.wait()
        s1.wait()
        sw0.wait()
        sw1.wait()

    return k(x, d0, d1, w0, w1)


def _ffn_body(te_ref, xs_ref, w1_ref, b1_ref, w2_ref, b2_ref, rw_ref, out_ref):
    h = jnp.dot(xs_ref[...], w1_ref[0], preferred_element_type=jnp.float32)
    h = jnp.maximum(h + b1_ref[0], 0.0)
    y = jnp.dot(h, w2_ref[0], preferred_element_type=jnp.float32)
    y = y + b2_ref[0]
    out_ref[...] = y * rw_ref[...][:, :1]


def _expert_ffn(xs, tile_e, row_w, W1, b1, W2, b2):
    grid_spec = pltpu.PrefetchScalarGridSpec(
        num_scalar_prefetch=1,
        grid=(NT,),
        in_specs=[
            pl.BlockSpec((BM, D_IN), lambda i, te: (i, 0)),
            pl.BlockSpec((1, D_IN, D_FF), lambda i, te: (te[i], 0, 0)),
            pl.BlockSpec((1, 1, D_FF), lambda i, te: (te[i], 0, 0)),
            pl.BlockSpec((1, D_FF, D_OUT), lambda i, te: (te[i], 0, 0)),
            pl.BlockSpec((1, 1, D_OUT), lambda i, te: (te[i], 0, 0)),
            pl.BlockSpec((BM, 128), lambda i, te: (i, 0)),
        ],
        out_specs=pl.BlockSpec((BM, D_OUT), lambda i, te: (i, 0)),
    )
    return pl.pallas_call(
        _ffn_body,
        grid_spec=grid_spec,
        out_shape=jax.ShapeDtypeStruct((R_MM, D_OUT), jnp.float32),
        compiler_params=pltpu.CompilerParams(
            dimension_semantics=("arbitrary",)),
    )(tile_e, xs, W1, b1, W2, b2, row_w)


def _combine(ys, pos0, pos1):
    info = plsc.get_sparse_core_info()
    nw = info.num_cores * info.num_subcores
    toks_pw = T // nw
    nch = 4
    chunk = toks_pw // nch
    mesh = plsc.VectorSubcoreMesh(core_axis_name="c", subcore_axis_name="s")

    @functools.partial(
        pl.kernel, mesh=mesh,
        out_type=jax.ShapeDtypeStruct((T, D_OUT), jnp.float32),
        scratch_types=[
            pltpu.VMEM((nch, chunk), jnp.int32),
            pltpu.VMEM((nch, chunk), jnp.int32),
            pltpu.VMEM((chunk, D_OUT), jnp.float32),
            pltpu.VMEM((chunk, D_OUT), jnp.float32),
            pltpu.VMEM((chunk, D_OUT), jnp.float32),
            pltpu.VMEM((chunk, D_OUT), jnp.float32),
            pltpu.SemaphoreType.DMA,
            pltpu.SemaphoreType.DMA,
        ],
    )
    def k(ys_hbm, p0_hbm, p1_hbm, out_hbm, i0_v, i1_v,
          r0a, r1a, r0b, r1b, sem_a, sem_b):
        wid = lax.axis_index("s") * info.num_cores + lax.axis_index("c")
        base = wid * toks_pw
        for ci in range(nch):
            pltpu.sync_copy(p0_hbm.at[pl.ds(base + ci * chunk, chunk)],
                            i0_v.at[ci])
            pltpu.sync_copy(p1_hbm.at[pl.ds(base + ci * chunk, chunk)],
                            i1_v.at[ci])
        bufs = ((r0a, r1a, sem_a), (r0b, r1b, sem_b))
        copies = [None] * (2 * nch)

        def fire(ci):
            b0, b1, sem = bufs[ci % 2]
            copies[2 * ci] = pltpu.async_copy(ys_hbm.at[i0_v.at[ci]], b0, sem)
            copies[2 * ci + 1] = pltpu.async_copy(ys_hbm.at[i1_v.at[ci]], b1, sem)

        fire(0)
        fire(1)
        for ci in range(nch):
            b0, b1, _ = bufs[ci % 2]
            copies[2 * ci].wait()
            copies[2 * ci + 1].wait()

            def row_add(r, _):
                for c in range(D_OUT // 16):
                    sl = pl.ds(c * 16, 16)
                    b0[r, sl] = b0[r, sl] + b1[r, sl]
                return 0

            lax.fori_loop(0, chunk, row_add, 0)
            pltpu.sync_copy(b0, out_hbm.at[pl.ds(base + ci * chunk, chunk)])
            if ci + 2 < nch:
                fire(ci + 2)

    return k(ys, pos0, pos1)


def kernel(x, Wg, bg, W1, b1, W2, b2):
    w0b, w1b, dst, te = _route(x, Wg, bg)
    d0 = dst[:, 0]
    d1 = dst[:, 1]
    tile_e = te.reshape(NTE)[:NT]
    xs, rw = _dispatch(x, d0, d1, w0b, w1b)
    ys = _expert_ffn(xs, tile_e, rw,
                     W1, b1.reshape(NUM_EXPERTS, 1, D_FF),
                     W2, b2.reshape(NUM_EXPERTS, 1, D_OUT))
    out = _combine(ys, d0, d1)
    return out
